# baseline (speedup 1.0000x reference)
import functools

import jax
import jax.numpy as jnp
from jax import lax
from jax.experimental import pallas as pl
from jax.experimental.pallas import tpu as pltpu
from jax.experimental.pallas import tpu_sc as plsc

B = 128
ND = 8192
NT = 65536
L = 512
D_FEAT = 128
T_IN = 64
HID = 256


def _make_agg(N, E, C, seg=2048, bt=128):
    E16 = E // 16
    CAP = seg + bt
    CR = N // C
    CPC = C // 2
    SHARE = CR // 16
    NSEG = E16 // seg
    NB16 = bt // 16

    mesh = plsc.VectorSubcoreMesh(core_axis_name="c", subcore_axis_name="s")
    scratch = [
        pltpu.VMEM((seg,), jnp.int32),
        pltpu.VMEM((seg,), jnp.int32),
        pltpu.VMEM((seg,), jnp.float32),
        pltpu.VMEM((CAP,), jnp.int32),
        pltpu.VMEM((CAP,), jnp.int32),
        pltpu.VMEM((CAP,), jnp.float32),
        pltpu.VMEM((2, bt), jnp.int32),
        pltpu.VMEM((bt, 128), jnp.float32),
        pltpu.VMEM_SHARED((CR, 128), jnp.float32),
        pltpu.SemaphoreType.DMA,
    ]

    @functools.partial(
        pl.kernel, mesh=mesh,
        out_type=jax.ShapeDtypeStruct((N, 128), jnp.float32),
        scratch_types=scratch,
        compiler_params=pltpu.CompilerParams(needs_layout_passes=False))
    def agg(table, src, dst, w, out, rdst, rsrc, rw, cdst, csrc, cw,
            idx2, rows, acc, sem):
        cid = lax.axis_index("c")
        sid = lax.axis_index("s")
        sbase = sid * E16

        def zero_vec16(ref, n):
            def zb(i, _):
                ref[pl.ds(i * 16, 16)] = jnp.zeros((16,), ref.dtype)
                return 0
            lax.fori_loop(0, n, zb, 0)

        def zero_rows(e, _):
            for r in range(8):
                rows[e, pl.ds(r * 16, 16)] = jnp.zeros((16,), jnp.float32)
            return 0

        for cidx in range(CPC):
            chunk = cidx * 2 + cid
            lo = chunk * CR
            lax.fori_loop(0, bt, zero_rows, 0)
            for q in range(SHARE // bt):
                pltpu.sync_copy(rows, acc.at[pl.ds(sid * SHARE + q * bt, bt)])
            zero_vec16(cdst, CAP // 16)
            zero_vec16(csrc, CAP // 16)
            plsc.subcore_barrier()

            def seg_body(g, _):
                off = sbase + g * seg
                pltpu.sync_copy(dst.at[pl.ds(off, seg)], rdst)
                pltpu.sync_copy(src.at[pl.ds(off, seg)], rsrc)
                pltpu.sync_copy(w.at[pl.ds(off, seg)], rw)
                zero_vec16(cw, CAP // 16)

                def scan_body(i, cnt):
                    d = rdst[pl.ds(i * 16, 16)]
                    s_ = rsrc[pl.ds(i * 16, 16)]
                    wv = rw[pl.ds(i * 16, 16)]
                    m = (d >= lo) & (d < lo + CR)
                    mi = m.astype(jnp.int32)
                    pos = cnt + plsc.cumsum(mi) - mi
                    plsc.store_scatter(cdst, [pos], d - lo, mask=m)
                    plsc.store_scatter(csrc, [pos], s_, mask=m)
                    plsc.store_scatter(cw, [pos], wv, mask=m)
                    return cnt + jnp.sum(mi)
                cnt = lax.fori_loop(0, seg // 16, scan_body, jnp.int32(0))

                nb = (cnt + bt - 1) // bt

                def proc(j, _):
                    base = j * bt
                    for k in range(NB16):
                        idx2[0, pl.ds(k * 16, 16)] = csrc[pl.ds(base + k * 16, 16)]
                        idx2[1, pl.ds(k * 16, 16)] = cdst[pl.ds(base + k * 16, 16)]
                    pltpu.async_copy(table.at[idx2.at[0]], rows, sem).wait()

                    def scale(e, _):
                        wb = plsc.load_gather(
                            cw, [jnp.broadcast_to(base + e, (16,)).astype(jnp.int32)])
                        for r in range(8):
                            rows[e, pl.ds(r * 16, 16)] = rows[e, pl.ds(r * 16, 16)] * wb
                        return 0
                    lax.fori_loop(0, bt, scale, 0)
                    pltpu.sync_copy(rows, acc.at[idx2.at[1]], add=True)
                    return 0

                lax.fori_loop(0, nb, proc, 0)
                return 0

            lax.fori_loop(0, NSEG, seg_body, 0)
            plsc.subcore_barrier()
            pltpu.sync_copy(acc.at[pl.ds(sid * SHARE, SHARE)],
                            out.at[pl.ds(lo + sid * SHARE, SHARE)])

    return agg


_agg_target = _make_agg(NT, 524288, 8)
_agg_drug = _make_agg(ND, 32768, 2)


def _normalize(x, eps=1e-12):
    n = jnp.sqrt(jnp.sum(x * x, axis=-1, keepdims=True))
    return x / jnp.maximum(n, eps)


BF = 8


def _fusion_body(t_seq_ref, t_feats_ref, t_agg_ref, w_t_ref, fc1w_ref, fc1b_ref,
                 w_t1_ref, fc2w_ref, fc2b_ref, op_ref):
    ts = t_seq_ref[...].reshape(BF * L, T_IN)
    tx = jnp.dot(ts, w_t_ref[...], preferred_element_type=jnp.float32)
    x1 = jnp.dot(tx, fc1w_ref[...], preferred_element_type=jnp.float32) + fc1b_ref[...]
    tf = (t_feats_ref[...] + t_agg_ref[...]).reshape(BF * L, D_FEAT)
    tx1 = jax.nn.relu(jnp.dot(tf, w_t1_ref[...], preferred_element_type=jnp.float32))
    x2 = jnp.dot(tx1, fc2w_ref[...], preferred_element_type=jnp.float32) + fc2b_ref[...]
    xco = (_normalize(x1) + _normalize(x2)).reshape(BF, L, HID)
    op_ref[...] = jnp.max(xco, axis=1)


def _fusion(T_seq, T_feats, t_agg, W_T, fc1_W, fc1_b, W_T1, fc2_W, fc2_b):
    grid = (B // BF,)
    return pl.pallas_call(
        _fusion_body,
        grid=grid,
        in_specs=[
            pl.BlockSpec((BF, L, T_IN), lambda b: (b, 0, 0)),
            pl.BlockSpec((BF, L, D_FEAT), lambda b: (b, 0, 0)),
            pl.BlockSpec((BF, L, D_FEAT), lambda b: (b, 0, 0)),
            pl.BlockSpec((T_IN, 128), lambda b: (0, 0)),
            pl.BlockSpec((128, HID), lambda b: (0, 0)),
            pl.BlockSpec((1, HID), lambda b: (0, 0)),
            pl.BlockSpec((D_FEAT, 128), lambda b: (0, 0)),
            pl.BlockSpec((128, HID), lambda b: (0, 0)),
            pl.BlockSpec((1, HID), lambda b: (0, 0)),
        ],
        out_specs=pl.BlockSpec((BF, HID), lambda b: (b, 0)),
        out_shape=jax.ShapeDtypeStruct((B, HID), jnp.float32),
    )(T_seq, T_feats, t_agg, W_T, fc1_W, fc1_b.reshape(1, HID),
      W_T1, fc2_W, fc2_b.reshape(1, HID))


def _hdot(a, b):
    return jax.lax.dot(a, b, precision=jax.lax.Precision.HIGHEST,
                       preferred_element_type=jnp.float32)


def _head_body(d_feats_ref, d_agg_ref, ids_ref, wd_ref, op_ref,
               p1w_ref, p1b_ref, p2w_ref, p2b_ref, p3w_ref, p3b_ref,
               fcw_ref, fcb_ref, out_ref):
    hD = jax.nn.relu(jnp.dot(d_feats_ref[...] + d_agg_ref[...], wd_ref[...],
                             preferred_element_type=jnp.float32))
    ids = ids_ref[...]
    seg = jax.lax.broadcasted_iota(jnp.int32, (B, ND), 0)
    mask = (seg == ids).astype(jnp.float32)
    g_sum = _hdot(mask, hD)
    g_cnt = jnp.sum(mask, axis=1, keepdims=True)
    d_x = g_sum / jnp.maximum(g_cnt, 1.0)
    dt = jnp.concatenate([d_x, op_ref[...]], axis=-1)
    h = jax.nn.relu(jnp.dot(dt, p1w_ref[...], preferred_element_type=jnp.float32) + p1b_ref[...])
    h = jax.nn.relu(jnp.dot(h, p2w_ref[...], preferred_element_type=jnp.float32) + p2b_ref[...])
    h = jnp.dot(h, p3w_ref[...], preferred_element_type=jnp.float32) + p3b_ref[...]
    out_ref[...] = jnp.dot(h, fcw_ref[...], preferred_element_type=jnp.float32) + fcb_ref[...]


def _head(D_feats, d_agg, ids_i32, W_D, op, p1_W, p1_b, p2_W, p2_b, p3_W, p3_b, fc_W, fc_b):
    return pl.pallas_call(
        _head_body,
        out_shape=jax.ShapeDtypeStruct((B, 2), jnp.float32),
    )(D_feats, d_agg, ids_i32.reshape(1, ND), W_D, op,
      p1_W, p1_b.reshape(1, -1), p2_W, p2_b.reshape(1, -1),
      p3_W, p3_b.reshape(1, -1), fc_W, fc_b.reshape(1, -1))


def kernel(D_feats, D_edge_index, D_graph_ids, T_feats, T_edge_index, T_edge_weight,
           T_seq, W_D, W_T1, W_T, fc1_W, fc1_b, fc2_W, fc2_b,
           p1_W, p1_b, p2_W, p2_b, p3_W, p3_b, fc_W, fc_b):
    d_src = D_edge_index[0].astype(jnp.int32)
    d_dst = D_edge_index[1].astype(jnp.int32)
    t_src = T_edge_index[0].astype(jnp.int32)
    t_dst = T_edge_index[1].astype(jnp.int32)
    ids = D_graph_ids.astype(jnp.int32)

    d_agg = _agg_drug(D_feats, d_src, d_dst, jnp.ones((d_src.shape[0],), jnp.float32))
    t_agg = _agg_target(T_feats, t_src, t_dst, T_edge_weight)

    op = _fusion(T_seq, T_feats.reshape(B, L, D_FEAT), t_agg.reshape(B, L, D_FEAT),
                 W_T, fc1_W, fc1_b, W_T1, fc2_W, fc2_b)
    out = _head(D_feats, d_agg, ids, W_D, op,
                p1_W, p1_b, p2_W, p2_b, p3_W, p3_b, fc_W, fc_b)
    return out


# speedup vs baseline: 1.8240x; 1.8240x over previous
import functools

import jax
import jax.numpy as jnp
from jax import lax
from jax.experimental import pallas as pl
from jax.experimental.pallas import tpu as pltpu
from jax.experimental.pallas import tpu_sc as plsc

B = 128
ND = 8192
NT = 65536
L = 512
D_FEAT = 128
T_IN = 64
HID = 256


def _make_agg(N, E, C, seg=2048, bt=128, table16=False):
    E16 = E // 16
    CAP = seg + bt
    CR = N // C
    CPC = C // 2
    SHARE = CR // 16
    NSEG = E16 // seg
    NB16 = bt // 16

    mesh = plsc.VectorSubcoreMesh(core_axis_name="c", subcore_axis_name="s")
    scratch = [
        pltpu.VMEM((seg,), jnp.int32),
        pltpu.VMEM((seg,), jnp.int32),
        pltpu.VMEM((seg,), jnp.float32),
        pltpu.VMEM((CAP,), jnp.int32),
        pltpu.VMEM((CAP,), jnp.int32),
        pltpu.VMEM((CAP,), jnp.float32),
        pltpu.VMEM((2, bt), jnp.int32),
        pltpu.VMEM((bt, 128), jnp.float32),
        pltpu.VMEM_SHARED((CR, 128), jnp.float32),
        pltpu.SemaphoreType.DMA,
    ]
    if table16:
        scratch.append(pltpu.VMEM((bt, 64), jnp.int32))

    @functools.partial(
        pl.kernel, mesh=mesh,
        out_type=jax.ShapeDtypeStruct((N, 128), jnp.float32),
        scratch_types=scratch,
        compiler_params=pltpu.CompilerParams(needs_layout_passes=False))
    def agg(table, src, dst, w, out, rdst, rsrc, rw, cdst, csrc, cw,
            idx2, rows, acc, sem, *maybe_rows16):
        rows16 = maybe_rows16[0] if table16 else None
        cid = lax.axis_index("c")
        sid = lax.axis_index("s")
        sbase = sid * E16

        def zero_vec16(ref, n):
            def zb(i, _):
                ref[pl.ds(i * 16, 16)] = jnp.zeros((16,), ref.dtype)
                return 0
            lax.fori_loop(0, n, zb, 0)

        def zero_rows(e, _):
            for r in range(8):
                rows[e, pl.ds(r * 16, 16)] = jnp.zeros((16,), jnp.float32)
            return 0

        for cidx in range(CPC):
            chunk = cidx * 2 + cid
            lo = chunk * CR
            lax.fori_loop(0, bt, zero_rows, 0)
            for q in range(SHARE // bt):
                pltpu.sync_copy(rows, acc.at[pl.ds(sid * SHARE + q * bt, bt)])
            zero_vec16(cdst, CAP // 16)
            zero_vec16(csrc, CAP // 16)
            plsc.subcore_barrier()

            def seg_body(g, _):
                off = sbase + g * seg
                c1 = pltpu.async_copy(dst.at[pl.ds(off, seg)], rdst, sem)
                c2 = pltpu.async_copy(src.at[pl.ds(off, seg)], rsrc, sem)
                c3 = pltpu.async_copy(w.at[pl.ds(off, seg)], rw, sem)
                c1.wait(); c2.wait(); c3.wait()
                zero_vec16(cw, CAP // 16)

                def scan_body(i, cnt):
                    d = rdst[pl.ds(i * 16, 16)]
                    s_ = rsrc[pl.ds(i * 16, 16)]
                    wv = rw[pl.ds(i * 16, 16)]
                    m = (d >= lo) & (d < lo + CR)
                    mi = m.astype(jnp.int32)
                    pos = cnt + plsc.cumsum(mi) - mi
                    plsc.store_scatter(cdst, [pos], d - lo, mask=m)
                    plsc.store_scatter(csrc, [pos], s_, mask=m)
                    plsc.store_scatter(cw, [pos], wv, mask=m)
                    return cnt + jnp.sum(mi)
                cnt = lax.fori_loop(0, seg // 16, scan_body, jnp.int32(0))

                nb = (cnt + bt - 1) // bt

                def proc(j, _):
                    base = j * bt
                    for k in range(NB16):
                        idx2[0, pl.ds(k * 16, 16)] = csrc[pl.ds(base + k * 16, 16)]
                        idx2[1, pl.ds(k * 16, 16)] = cdst[pl.ds(base + k * 16, 16)]
                    if table16:
                        pltpu.async_copy(table.at[idx2.at[0]], rows16, sem).wait()
                        even = jax.lax.iota(jnp.int32, 16) * 2

                        def scale(e, _):
                            wb = plsc.load_gather(
                                cw, [jnp.broadcast_to(base + e, (16,)).astype(jnp.int32)])
                            row = rows.at[e]
                            for c in range(4):
                                x = plsc.bitcast(rows16[e, pl.ds(c * 16, 16)],
                                                 jnp.bfloat16)
                                a, b2 = plsc.unpack(x, format=plsc.PackFormat.INTERLEAVED)
                                plsc.store_scatter(row, [c * 32 + even], a * wb)
                                plsc.store_scatter(row, [c * 32 + even + 1], b2 * wb)
                            return 0
                        lax.fori_loop(0, bt, scale, 0)
                    else:
                        pltpu.async_copy(table.at[idx2.at[0]], rows, sem).wait()

                        def scale(e, _):
                            wb = plsc.load_gather(
                                cw, [jnp.broadcast_to(base + e, (16,)).astype(jnp.int32)])
                            for r in range(8):
                                rows[e, pl.ds(r * 16, 16)] = rows[e, pl.ds(r * 16, 16)] * wb
                            return 0
                        lax.fori_loop(0, bt, scale, 0)
                    pltpu.sync_copy(rows, acc.at[idx2.at[1]], add=True)
                    return 0

                lax.fori_loop(0, nb, proc, 0)
                return 0

            lax.fori_loop(0, NSEG, seg_body, 0)
            plsc.subcore_barrier()
            pltpu.sync_copy(acc.at[pl.ds(sid * SHARE, SHARE)],
                            out.at[pl.ds(lo + sid * SHARE, SHARE)])

    return agg


_agg_target = _make_agg(NT, 524288, 8, seg=4096)
_agg_drug = _make_agg(ND, 32768, 2)


def _normalize(x, eps=1e-12):
    n = jnp.sqrt(jnp.sum(x * x, axis=-1, keepdims=True))
    return x / jnp.maximum(n, eps)


BF = 8


def _fusion_body(t_seq_ref, t_feats_ref, t_agg_ref, w_t_ref, fc1w_ref, fc1b_ref,
                 w_t1_ref, fc2w_ref, fc2b_ref, op_ref):
    ts = t_seq_ref[...].reshape(BF * L, T_IN)
    tx = jnp.dot(ts, w_t_ref[...], preferred_element_type=jnp.float32)
    x1 = jnp.dot(tx, fc1w_ref[...], preferred_element_type=jnp.float32) + fc1b_ref[...]
    tf = (t_feats_ref[...] + t_agg_ref[...]).reshape(BF * L, D_FEAT)
    tx1 = jax.nn.relu(jnp.dot(tf, w_t1_ref[...], preferred_element_type=jnp.float32))
    x2 = jnp.dot(tx1, fc2w_ref[...], preferred_element_type=jnp.float32) + fc2b_ref[...]
    xco = (_normalize(x1) + _normalize(x2)).reshape(BF, L, HID)
    op_ref[...] = jnp.max(xco, axis=1)


def _fusion(T_seq, T_feats, t_agg, W_T, fc1_W, fc1_b, W_T1, fc2_W, fc2_b):
    grid = (B // BF,)
    return pl.pallas_call(
        _fusion_body,
        grid=grid,
        in_specs=[
            pl.BlockSpec((BF, L, T_IN), lambda b: (b, 0, 0)),
            pl.BlockSpec((BF, L, D_FEAT), lambda b: (b, 0, 0)),
            pl.BlockSpec((BF, L, D_FEAT), lambda b: (b, 0, 0)),
            pl.BlockSpec((T_IN, 128), lambda b: (0, 0)),
            pl.BlockSpec((128, HID), lambda b: (0, 0)),
            pl.BlockSpec((1, HID), lambda b: (0, 0)),
            pl.BlockSpec((D_FEAT, 128), lambda b: (0, 0)),
            pl.BlockSpec((128, HID), lambda b: (0, 0)),
            pl.BlockSpec((1, HID), lambda b: (0, 0)),
        ],
        out_specs=pl.BlockSpec((BF, HID), lambda b: (b, 0)),
        out_shape=jax.ShapeDtypeStruct((B, HID), jnp.float32),
    )(T_seq, T_feats, t_agg, W_T, fc1_W, fc1_b.reshape(1, HID),
      W_T1, fc2_W, fc2_b.reshape(1, HID))


def _hdot(a, b):
    return jax.lax.dot(a, b, precision=jax.lax.Precision.HIGHEST,
                       preferred_element_type=jnp.float32)


def _head_body(d_feats_ref, d_agg_ref, ids_ref, wd_ref, op_ref,
               p1w_ref, p1b_ref, p2w_ref, p2b_ref, p3w_ref, p3b_ref,
               fcw_ref, fcb_ref, out_ref):
    hD = jax.nn.relu(jnp.dot(d_feats_ref[...] + d_agg_ref[...], wd_ref[...],
                             preferred_element_type=jnp.float32))
    ids = ids_ref[...]
    seg = jax.lax.broadcasted_iota(jnp.int32, (B, ND), 0)
    mask = (seg == ids).astype(jnp.float32)
    g_sum = _hdot(mask, hD)
    g_cnt = jnp.sum(mask, axis=1, keepdims=True)
    d_x = g_sum / jnp.maximum(g_cnt, 1.0)
    dt = jnp.concatenate([d_x, op_ref[...]], axis=-1)
    h = jax.nn.relu(jnp.dot(dt, p1w_ref[...], preferred_element_type=jnp.float32) + p1b_ref[...])
    h = jax.nn.relu(jnp.dot(h, p2w_ref[...], preferred_element_type=jnp.float32) + p2b_ref[...])
    h = jnp.dot(h, p3w_ref[...], preferred_element_type=jnp.float32) + p3b_ref[...]
    out_ref[...] = jnp.dot(h, fcw_ref[...], preferred_element_type=jnp.float32) + fcb_ref[...]


def _head(D_feats, d_agg, ids_i32, W_D, op, p1_W, p1_b, p2_W, p2_b, p3_W, p3_b, fc_W, fc_b):
    return pl.pallas_call(
        _head_body,
        out_shape=jax.ShapeDtypeStruct((B, 2), jnp.float32),
    )(D_feats, d_agg, ids_i32.reshape(1, ND), W_D, op,
      p1_W, p1_b.reshape(1, -1), p2_W, p2_b.reshape(1, -1),
      p3_W, p3_b.reshape(1, -1), fc_W, fc_b.reshape(1, -1))


def kernel(D_feats, D_edge_index, D_graph_ids, T_feats, T_edge_index, T_edge_weight,
           T_seq, W_D, W_T1, W_T, fc1_W, fc1_b, fc2_W, fc2_b,
           p1_W, p1_b, p2_W, p2_b, p3_W, p3_b, fc_W, fc_b):
    d_src = D_edge_index[0].astype(jnp.int32)
    d_dst = D_edge_index[1].astype(jnp.int32)
    t_src = T_edge_index[0].astype(jnp.int32)
    t_dst = T_edge_index[1].astype(jnp.int32)
    ids = D_graph_ids.astype(jnp.int32)

    d_agg = _agg_drug(D_feats, d_src, d_dst, jnp.ones((d_src.shape[0],), jnp.float32))
    t_agg = _agg_target(T_feats, t_src, t_dst, T_edge_weight)

    op = _fusion(T_seq, T_feats.reshape(B, L, D_FEAT), t_agg.reshape(B, L, D_FEAT),
                 W_T, fc1_W, fc1_b, W_T1, fc2_W, fc2_b)
    out = _head(D_feats, d_agg, ids, W_D, op,
                p1_W, p1_b, p2_W, p2_b, p3_W, p3_b, fc_W, fc_b)
    return out


# speedup vs baseline: 1.8318x; 1.0043x over previous
import functools

import jax
import jax.numpy as jnp
from jax import lax
from jax.experimental import pallas as pl
from jax.experimental.pallas import tpu as pltpu
from jax.experimental.pallas import tpu_sc as plsc

B = 128
ND = 8192
NT = 65536
L = 512
D_FEAT = 128
T_IN = 64
HID = 256


def _make_agg(N, E, C, seg=2048, bt=128, table16=False):
    E16 = E // 16
    CAP = seg + bt
    CR = N // C
    CPC = C // 2
    SHARE = CR // 16
    NSEG = E16 // seg
    NB16 = bt // 16

    mesh = plsc.VectorSubcoreMesh(core_axis_name="c", subcore_axis_name="s")
    scratch = [
        pltpu.VMEM((seg,), jnp.int32),
        pltpu.VMEM((seg,), jnp.int32),
        pltpu.VMEM((seg,), jnp.float32),
        pltpu.VMEM((CAP,), jnp.int32),
        pltpu.VMEM((CAP,), jnp.int32),
        pltpu.VMEM((CAP,), jnp.float32),
        pltpu.VMEM((2, bt), jnp.int32),
        pltpu.VMEM((2, bt), jnp.int32),
        pltpu.VMEM((bt, 128), jnp.float32),
        pltpu.VMEM((bt, 128), jnp.float32),
        pltpu.VMEM_SHARED((CR, 128), jnp.float32),
        pltpu.SemaphoreType.DMA((3,)),
    ]

    @functools.partial(
        pl.kernel, mesh=mesh,
        out_type=jax.ShapeDtypeStruct((N, 128), jnp.float32),
        scratch_types=scratch,
        compiler_params=pltpu.CompilerParams(needs_layout_passes=False))
    def agg(table, src, dst, w, out, rdst, rsrc, rw, cdst, csrc, cw,
            idxA, idxB, rowsA, rowsB, acc, sem):
        rows = rowsA
        cid = lax.axis_index("c")
        sid = lax.axis_index("s")
        sbase = sid * E16

        def raw_issue(g):
            off = sbase + g * seg
            pltpu.async_copy(dst.at[pl.ds(off, seg)], rdst, sem.at[0])
            pltpu.async_copy(src.at[pl.ds(off, seg)], rsrc, sem.at[0])
            pltpu.async_copy(w.at[pl.ds(off, seg)], rw, sem.at[0])

        def raw_wait(g):
            off = sbase + g * seg
            pltpu.make_async_copy(dst.at[pl.ds(off, seg)], rdst, sem.at[0]).wait()
            pltpu.make_async_copy(src.at[pl.ds(off, seg)], rsrc, sem.at[0]).wait()
            pltpu.make_async_copy(w.at[pl.ds(off, seg)], rw, sem.at[0]).wait()

        def zero_vec16(ref, n):
            def zb(i, _):
                ref[pl.ds(i * 16, 16)] = jnp.zeros((16,), ref.dtype)
                return 0
            lax.fori_loop(0, n, zb, 0)

        def zero_rows(e, _):
            for r in range(8):
                rows[e, pl.ds(r * 16, 16)] = jnp.zeros((16,), jnp.float32)
            return 0

        for cidx in range(CPC):
            chunk = cidx * 2 + cid
            lo = chunk * CR
            lax.fori_loop(0, bt, zero_rows, 0)
            for q in range(SHARE // bt):
                pltpu.sync_copy(rows, acc.at[pl.ds(sid * SHARE + q * bt, bt)])
            zero_vec16(cdst, CAP // 16)
            zero_vec16(csrc, CAP // 16)
            plsc.subcore_barrier()

            raw_issue(0)

            def seg_body(g, _):
                raw_wait(g)
                zero_vec16(cw, CAP // 16)

                def scan_body(i, cnt):
                    d = rdst[pl.ds(i * 16, 16)]
                    s_ = rsrc[pl.ds(i * 16, 16)]
                    wv = rw[pl.ds(i * 16, 16)]
                    m = (d >= lo) & (d < lo + CR)
                    mi = m.astype(jnp.int32)
                    pos = cnt + plsc.cumsum(mi) - mi
                    plsc.store_scatter(cdst, [pos], d - lo, mask=m)
                    plsc.store_scatter(csrc, [pos], s_, mask=m)
                    plsc.store_scatter(cw, [pos], wv, mask=m)
                    return cnt + jnp.sum(mi)
                cnt = lax.fori_loop(0, seg // 16, scan_body, jnp.int32(0))

                @pl.when(g + 1 < NSEG)
                def _():
                    raw_issue(g + 1)

                nb = (cnt + bt - 1) // bt
                bufs = ((idxA, rowsA, 1), (idxB, rowsB, 2))

                def stage(j, ib):
                    base = j * bt
                    for k in range(NB16):
                        ib[0, pl.ds(k * 16, 16)] = csrc[pl.ds(base + k * 16, 16)]
                        ib[1, pl.ds(k * 16, 16)] = cdst[pl.ds(base + k * 16, 16)]

                def scale_scatter(j, ib, rb):
                    base = j * bt

                    def scale(e, _):
                        wb = plsc.load_gather(
                            cw, [jnp.broadcast_to(base + e, (16,)).astype(jnp.int32)])
                        for r in range(8):
                            rb[e, pl.ds(r * 16, 16)] = rb[e, pl.ds(r * 16, 16)] * wb
                        return 0
                    lax.fori_loop(0, bt, scale, 0)
                    pltpu.sync_copy(rb, acc.at[ib.at[1]], add=True)

                @pl.when(nb > 0)
                def _():
                    stage(0, idxA)
                    pltpu.async_copy(table.at[idxA.at[0]], rowsA, sem.at[1])

                def pair(jj, _):
                    for par in range(2):
                        j = 2 * jj + par
                        ib, rb, sid_ = bufs[par]
                        nib, nrb, nsid = bufs[1 - par]

                        @pl.when(j < nb)
                        def _():
                            @pl.when(j + 1 < nb)
                            def _():
                                stage(j + 1, nib)
                                pltpu.async_copy(table.at[nib.at[0]], nrb,
                                                 sem.at[nsid])
                            pltpu.make_async_copy(table.at[ib.at[0]], rb,
                                                  sem.at[sid_]).wait()
                            scale_scatter(j, ib, rb)
                    return 0

                lax.fori_loop(0, (nb + 1) // 2, pair, 0)
                return 0

            lax.fori_loop(0, NSEG, seg_body, 0)
            plsc.subcore_barrier()
            pltpu.sync_copy(acc.at[pl.ds(sid * SHARE, SHARE)],
                            out.at[pl.ds(lo + sid * SHARE, SHARE)])

    return agg


_agg_target = _make_agg(NT, 524288, 8, seg=4096)
_agg_drug = _make_agg(ND, 32768, 2)


def _normalize(x, eps=1e-12):
    n = jnp.sqrt(jnp.sum(x * x, axis=-1, keepdims=True))
    return x / jnp.maximum(n, eps)


BF = 8


def _fusion_body(t_seq_ref, t_feats_ref, t_agg_ref, w_t_ref, fc1w_ref, fc1b_ref,
                 w_t1_ref, fc2w_ref, fc2b_ref, op_ref):
    ts = t_seq_ref[...].reshape(BF * L, T_IN)
    tx = jnp.dot(ts, w_t_ref[...], preferred_element_type=jnp.float32)
    x1 = jnp.dot(tx, fc1w_ref[...], preferred_element_type=jnp.float32) + fc1b_ref[...]
    tf = (t_feats_ref[...] + t_agg_ref[...]).reshape(BF * L, D_FEAT)
    tx1 = jax.nn.relu(jnp.dot(tf, w_t1_ref[...], preferred_element_type=jnp.float32))
    x2 = jnp.dot(tx1, fc2w_ref[...], preferred_element_type=jnp.float32) + fc2b_ref[...]
    xco = (_normalize(x1) + _normalize(x2)).reshape(BF, L, HID)
    op_ref[...] = jnp.max(xco, axis=1)


def _fusion(T_seq, T_feats, t_agg, W_T, fc1_W, fc1_b, W_T1, fc2_W, fc2_b):
    grid = (B // BF,)
    return pl.pallas_call(
        _fusion_body,
        grid=grid,
        in_specs=[
            pl.BlockSpec((BF, L, T_IN), lambda b: (b, 0, 0)),
            pl.BlockSpec((BF, L, D_FEAT), lambda b: (b, 0, 0)),
            pl.BlockSpec((BF, L, D_FEAT), lambda b: (b, 0, 0)),
            pl.BlockSpec((T_IN, 128), lambda b: (0, 0)),
            pl.BlockSpec((128, HID), lambda b: (0, 0)),
            pl.BlockSpec((1, HID), lambda b: (0, 0)),
            pl.BlockSpec((D_FEAT, 128), lambda b: (0, 0)),
            pl.BlockSpec((128, HID), lambda b: (0, 0)),
            pl.BlockSpec((1, HID), lambda b: (0, 0)),
        ],
        out_specs=pl.BlockSpec((BF, HID), lambda b: (b, 0)),
        out_shape=jax.ShapeDtypeStruct((B, HID), jnp.float32),
    )(T_seq, T_feats, t_agg, W_T, fc1_W, fc1_b.reshape(1, HID),
      W_T1, fc2_W, fc2_b.reshape(1, HID))


def _hdot(a, b):
    return jax.lax.dot(a, b, precision=jax.lax.Precision.HIGHEST,
                       preferred_element_type=jnp.float32)


def _head_body(d_feats_ref, d_agg_ref, ids_ref, wd_ref, op_ref,
               p1w_ref, p1b_ref, p2w_ref, p2b_ref, p3w_ref, p3b_ref,
               fcw_ref, fcb_ref, out_ref):
    hD = jax.nn.relu(jnp.dot(d_feats_ref[...] + d_agg_ref[...], wd_ref[...],
                             preferred_element_type=jnp.float32))
    ids = ids_ref[...]
    seg = jax.lax.broadcasted_iota(jnp.int32, (B, ND), 0)
    mask = (seg == ids).astype(jnp.float32)
    g_sum = _hdot(mask, hD)
    g_cnt = jnp.sum(mask, axis=1, keepdims=True)
    d_x = g_sum / jnp.maximum(g_cnt, 1.0)
    dt = jnp.concatenate([d_x, op_ref[...]], axis=-1)
    h = jax.nn.relu(jnp.dot(dt, p1w_ref[...], preferred_element_type=jnp.float32) + p1b_ref[...])
    h = jax.nn.relu(jnp.dot(h, p2w_ref[...], preferred_element_type=jnp.float32) + p2b_ref[...])
    h = jnp.dot(h, p3w_ref[...], preferred_element_type=jnp.float32) + p3b_ref[...]
    out_ref[...] = jnp.dot(h, fcw_ref[...], preferred_element_type=jnp.float32) + fcb_ref[...]


def _head(D_feats, d_agg, ids_i32, W_D, op, p1_W, p1_b, p2_W, p2_b, p3_W, p3_b, fc_W, fc_b):
    return pl.pallas_call(
        _head_body,
        out_shape=jax.ShapeDtypeStruct((B, 2), jnp.float32),
    )(D_feats, d_agg, ids_i32.reshape(1, ND), W_D, op,
      p1_W, p1_b.reshape(1, -1), p2_W, p2_b.reshape(1, -1),
      p3_W, p3_b.reshape(1, -1), fc_W, fc_b.reshape(1, -1))


def kernel(D_feats, D_edge_index, D_graph_ids, T_feats, T_edge_index, T_edge_weight,
           T_seq, W_D, W_T1, W_T, fc1_W, fc1_b, fc2_W, fc2_b,
           p1_W, p1_b, p2_W, p2_b, p3_W, p3_b, fc_W, fc_b):
    d_src = D_edge_index[0].astype(jnp.int32)
    d_dst = D_edge_index[1].astype(jnp.int32)
    t_src = T_edge_index[0].astype(jnp.int32)
    t_dst = T_edge_index[1].astype(jnp.int32)
    ids = D_graph_ids.astype(jnp.int32)

    d_agg = _agg_drug(D_feats, d_src, d_dst, jnp.ones((d_src.shape[0],), jnp.float32))
    t_agg = _agg_target(T_feats, t_src, t_dst, T_edge_weight)

    op = _fusion(T_seq, T_feats.reshape(B, L, D_FEAT), t_agg.reshape(B, L, D_FEAT),
                 W_T, fc1_W, fc1_b, W_T1, fc2_W, fc2_b)
    out = _head(D_feats, d_agg, ids, W_D, op,
                p1_W, p1_b, p2_W, p2_b, p3_W, p3_b, fc_W, fc_b)
    return out


# speedup vs baseline: 1.8374x; 1.0030x over previous
import functools

import jax
import jax.numpy as jnp
from jax import lax
from jax.experimental import pallas as pl
from jax.experimental.pallas import tpu as pltpu
from jax.experimental.pallas import tpu_sc as plsc

B = 128
ND = 8192
NT = 65536
L = 512
D_FEAT = 128
T_IN = 64
HID = 256


def _make_agg(N, E, C, seg=2048, bt=128):
    E16 = E // 16
    CAP = seg + bt
    CR = N // C
    CPC = C // 2
    SHARE = CR // 16
    NSEG = E16 // seg
    NB16 = bt // 16

    mesh = plsc.VectorSubcoreMesh(core_axis_name="c", subcore_axis_name="s")
    scratch = [
        pltpu.VMEM((seg,), jnp.int32),
        pltpu.VMEM((seg,), jnp.int32),
        pltpu.VMEM((seg,), jnp.float32),
        pltpu.VMEM((CAP,), jnp.int32),
        pltpu.VMEM((CAP,), jnp.int32),
        pltpu.VMEM((CAP,), jnp.float32),
        pltpu.VMEM((2, bt), jnp.int32),
        pltpu.VMEM((2, bt), jnp.int32),
        pltpu.VMEM((bt, 128), jnp.float32),
        pltpu.VMEM((bt, 128), jnp.float32),
        pltpu.VMEM_SHARED((CR, 128), jnp.float32),
        pltpu.SemaphoreType.DMA((3,)),
    ]

    @functools.partial(
        pl.kernel, mesh=mesh,
        out_type=jax.ShapeDtypeStruct((N, 128), jnp.float32),
        scratch_types=scratch,
        compiler_params=pltpu.CompilerParams(needs_layout_passes=False))
    def agg(table, src, dst, w, out, rdst, rsrc, rw, cdst, csrc, cw,
            idxA, idxB, rowsA, rowsB, acc, sem):
        rows = rowsA
        cid = lax.axis_index("c")
        sid = lax.axis_index("s")
        sbase = sid * E16

        def raw_issue(g):
            off = sbase + g * seg
            pltpu.async_copy(dst.at[pl.ds(off, seg)], rdst, sem.at[0])
            pltpu.async_copy(src.at[pl.ds(off, seg)], rsrc, sem.at[0])
            pltpu.async_copy(w.at[pl.ds(off, seg)], rw, sem.at[0])

        def raw_wait(g):
            off = sbase + g * seg
            pltpu.make_async_copy(dst.at[pl.ds(off, seg)], rdst, sem.at[0]).wait()
            pltpu.make_async_copy(src.at[pl.ds(off, seg)], rsrc, sem.at[0]).wait()
            pltpu.make_async_copy(w.at[pl.ds(off, seg)], rw, sem.at[0]).wait()

        def zero_vec16(ref, n):
            def zb(i, _):
                ref[pl.ds(i * 16, 16)] = jnp.zeros((16,), ref.dtype)
                return 0
            lax.fori_loop(0, n, zb, 0)

        def zero_rows(e, _):
            for r in range(8):
                rows[e, pl.ds(r * 16, 16)] = jnp.zeros((16,), jnp.float32)
            return 0

        for cidx in range(CPC):
            chunk = cidx * 2 + cid
            lo = chunk * CR
            lax.fori_loop(0, bt, zero_rows, 0)
            for q in range(SHARE // bt):
                pltpu.sync_copy(rows, acc.at[pl.ds(sid * SHARE + q * bt, bt)])
            zero_vec16(cdst, CAP // 16)
            zero_vec16(csrc, CAP // 16)
            plsc.subcore_barrier()

            raw_issue(0)

            def seg_body(g, _):
                raw_wait(g)
                zero_vec16(cw, CAP // 16)

                def scan_body(i, cnt):
                    d = rdst[pl.ds(i * 16, 16)]
                    s_ = rsrc[pl.ds(i * 16, 16)]
                    wv = rw[pl.ds(i * 16, 16)]
                    m = (d >= lo) & (d < lo + CR)
                    mi = m.astype(jnp.int32)
                    pos = cnt + plsc.cumsum(mi) - mi
                    plsc.store_scatter(cdst, [pos], d - lo, mask=m)
                    plsc.store_scatter(csrc, [pos], s_, mask=m)
                    plsc.store_scatter(cw, [pos], wv, mask=m)
                    return cnt + jnp.sum(mi)
                cnt = lax.fori_loop(0, seg // 16, scan_body, jnp.int32(0))

                @pl.when(g + 1 < NSEG)
                def _():
                    raw_issue(g + 1)

                nb = (cnt + bt - 1) // bt
                bufs = ((idxA, rowsA, 1), (idxB, rowsB, 2))

                def stage(j, ib):
                    base = j * bt
                    for k in range(NB16):
                        ib[0, pl.ds(k * 16, 16)] = csrc[pl.ds(base + k * 16, 16)]
                        ib[1, pl.ds(k * 16, 16)] = cdst[pl.ds(base + k * 16, 16)]

                def scale_scatter(j, ib, rb):
                    base = j * bt

                    def scale(e, _):
                        wb = plsc.load_gather(
                            cw, [jnp.broadcast_to(base + e, (16,)).astype(jnp.int32)])
                        for r in range(8):
                            rb[e, pl.ds(r * 16, 16)] = rb[e, pl.ds(r * 16, 16)] * wb
                        return 0
                    lax.fori_loop(0, bt, scale, 0)
                    pltpu.sync_copy(rb, acc.at[ib.at[1]], add=True)

                @pl.when(nb > 0)
                def _():
                    stage(0, idxA)
                    pltpu.async_copy(table.at[idxA.at[0]], rowsA, sem.at[1])

                def pair(jj, _):
                    for par in range(2):
                        j = 2 * jj + par
                        ib, rb, sid_ = bufs[par]
                        nib, nrb, nsid = bufs[1 - par]

                        @pl.when(j < nb)
                        def _():
                            @pl.when(j + 1 < nb)
                            def _():
                                stage(j + 1, nib)
                                pltpu.async_copy(table.at[nib.at[0]], nrb,
                                                 sem.at[nsid])
                            pltpu.make_async_copy(table.at[ib.at[0]], rb,
                                                  sem.at[sid_]).wait()
                            scale_scatter(j, ib, rb)
                    return 0

                lax.fori_loop(0, (nb + 1) // 2, pair, 0)
                return 0

            lax.fori_loop(0, NSEG, seg_body, 0)
            plsc.subcore_barrier()
            pltpu.sync_copy(acc.at[pl.ds(sid * SHARE, SHARE)],
                            out.at[pl.ds(lo + sid * SHARE, SHARE)])

    return agg


_agg_target = _make_agg(NT, 524288, 8, seg=4096)
_agg_drug = _make_agg(ND, 32768, 2)


def _normalize(x, eps=1e-12):
    n = jnp.sqrt(jnp.sum(x * x, axis=-1, keepdims=True))
    return x / jnp.maximum(n, eps)


BF = 8


def _fusion_body(t_seq_ref, t_feats_ref, t_agg_ref, w_t_ref, fc1w_ref, fc1b_ref,
                 w_t1_ref, fc2w_ref, fc2b_ref, op_ref):
    ts = t_seq_ref[...].reshape(BF * L, T_IN)
    tx = jnp.dot(ts, w_t_ref[...], preferred_element_type=jnp.float32)
    x1 = jnp.dot(tx, fc1w_ref[...], preferred_element_type=jnp.float32) + fc1b_ref[...]
    tf = (t_feats_ref[...] + t_agg_ref[...]).reshape(BF * L, D_FEAT)
    tx1 = jax.nn.relu(jnp.dot(tf, w_t1_ref[...], preferred_element_type=jnp.float32))
    x2 = jnp.dot(tx1, fc2w_ref[...], preferred_element_type=jnp.float32) + fc2b_ref[...]
    xco = (_normalize(x1) + _normalize(x2)).reshape(BF, L, HID)
    op_ref[...] = jnp.max(xco, axis=1)


def _fusion(T_seq, T_feats, t_agg, W_T, fc1_W, fc1_b, W_T1, fc2_W, fc2_b):
    grid = (B // BF,)
    return pl.pallas_call(
        _fusion_body,
        grid=grid,
        in_specs=[
            pl.BlockSpec((BF, L, T_IN), lambda b: (b, 0, 0)),
            pl.BlockSpec((BF, L, D_FEAT), lambda b: (b, 0, 0)),
            pl.BlockSpec((BF, L, D_FEAT), lambda b: (b, 0, 0)),
            pl.BlockSpec((T_IN, 128), lambda b: (0, 0)),
            pl.BlockSpec((128, HID), lambda b: (0, 0)),
            pl.BlockSpec((1, HID), lambda b: (0, 0)),
            pl.BlockSpec((D_FEAT, 128), lambda b: (0, 0)),
            pl.BlockSpec((128, HID), lambda b: (0, 0)),
            pl.BlockSpec((1, HID), lambda b: (0, 0)),
        ],
        out_specs=pl.BlockSpec((BF, HID), lambda b: (b, 0)),
        out_shape=jax.ShapeDtypeStruct((B, HID), jnp.float32),
    )(T_seq, T_feats, t_agg, W_T, fc1_W, fc1_b.reshape(1, HID),
      W_T1, fc2_W, fc2_b.reshape(1, HID))


def _hdot(a, b):
    return jax.lax.dot(a, b, precision=jax.lax.Precision.HIGHEST,
                       preferred_element_type=jnp.float32)


def _head_body(d_feats_ref, d_agg_ref, ids_ref, wd_ref, op_ref,
               p1w_ref, p1b_ref, p2w_ref, p2b_ref, p3w_ref, p3b_ref,
               fcw_ref, fcb_ref, out_ref):
    hD = jax.nn.relu(jnp.dot(d_feats_ref[...] + d_agg_ref[...], wd_ref[...],
                             preferred_element_type=jnp.float32))
    ids = ids_ref[...]
    seg = jax.lax.broadcasted_iota(jnp.int32, (B, ND), 0)
    mask = (seg == ids).astype(jnp.float32)
    g_sum = _hdot(mask, hD)
    g_cnt = jnp.sum(mask, axis=1, keepdims=True)
    d_x = g_sum / jnp.maximum(g_cnt, 1.0)
    dt = jnp.concatenate([d_x, op_ref[...]], axis=-1)
    h = jax.nn.relu(jnp.dot(dt, p1w_ref[...], preferred_element_type=jnp.float32) + p1b_ref[...])
    h = jax.nn.relu(jnp.dot(h, p2w_ref[...], preferred_element_type=jnp.float32) + p2b_ref[...])
    h = jnp.dot(h, p3w_ref[...], preferred_element_type=jnp.float32) + p3b_ref[...]
    out_ref[...] = jnp.dot(h, fcw_ref[...], preferred_element_type=jnp.float32) + fcb_ref[...]


def _head(D_feats, d_agg, ids_i32, W_D, op, p1_W, p1_b, p2_W, p2_b, p3_W, p3_b, fc_W, fc_b):
    return pl.pallas_call(
        _head_body,
        out_shape=jax.ShapeDtypeStruct((B, 2), jnp.float32),
    )(D_feats, d_agg, ids_i32.reshape(1, ND), W_D, op,
      p1_W, p1_b.reshape(1, -1), p2_W, p2_b.reshape(1, -1),
      p3_W, p3_b.reshape(1, -1), fc_W, fc_b.reshape(1, -1))


def kernel(D_feats, D_edge_index, D_graph_ids, T_feats, T_edge_index, T_edge_weight,
           T_seq, W_D, W_T1, W_T, fc1_W, fc1_b, fc2_W, fc2_b,
           p1_W, p1_b, p2_W, p2_b, p3_W, p3_b, fc_W, fc_b):
    d_src = D_edge_index[0].astype(jnp.int32)
    d_dst = D_edge_index[1].astype(jnp.int32)
    t_src = T_edge_index[0].astype(jnp.int32)
    t_dst = T_edge_index[1].astype(jnp.int32)
    ids = D_graph_ids.astype(jnp.int32)

    d_agg = _agg_drug(D_feats, d_src, d_dst, jnp.ones((d_src.shape[0],), jnp.float32))
    t_agg = _agg_target(T_feats, t_src, t_dst, T_edge_weight)

    op = _fusion(T_seq, T_feats.reshape(B, L, D_FEAT), t_agg.reshape(B, L, D_FEAT),
                 W_T, fc1_W, fc1_b, W_T1, fc2_W, fc2_b)
    out = _head(D_feats, d_agg, ids, W_D, op,
                p1_W, p1_b, p2_W, p2_b, p3_W, p3_b, fc_W, fc_b)
    return out
